# baseline (speedup 1.0000x reference)
import functools

import jax
import jax.numpy as jnp
from jax import lax
from jax.experimental import pallas as pl
from jax.experimental.pallas import tpu as pltpu
from jax.experimental.pallas import tpu_sc as plsc

N = 10000
E = 320000
F1 = 128
F2 = 40
F2P = 128
LANES = 16
NC, NS = 2, 16
NW = NC * NS
CHUNK = 64
EROWS = 5120
EPAD = EROWS * CHUNK - E
RPW = EROWS // NW
HRP = RPW // 4
RT = 632
RTL = N - (NS - 1) * RT
DEGS = 640
DEGP = NS * DEGS
ACCP = N + CHUNK

_mesh = plsc.VectorSubcoreMesh(core_axis_name="c", subcore_axis_name="s")


def _deg_body(dst_hbm, out_hbm, deg_sp, dst_blk, zb, ones_v):
    c = lax.axis_index("c")
    s = lax.axis_index("s")
    w = s * NC + c

    def fill(i, _):
        zb[pl.ds(i * LANES, LANES)] = jnp.zeros((LANES,), jnp.float32)
        return 0

    lax.fori_loop(0, DEGS // LANES, fill, 0)

    def fill1(i, _):
        ones_v[pl.ds(i * LANES, LANES)] = jnp.ones((LANES,), jnp.float32)
        return 0

    lax.fori_loop(0, CHUNK // LANES, fill1, 0)

    pltpu.sync_copy(zb, deg_sp.at[pl.ds(s * DEGS, DEGS)])
    pltpu.sync_copy(dst_hbm.at[pl.ds(w * RPW, RPW)], dst_blk)
    plsc.subcore_barrier()

    def body(j, _):
        pltpu.sync_copy(ones_v, deg_sp.at[dst_blk.at[j]], add=True)
        return 0

    lax.fori_loop(0, RPW, body, 0)
    plsc.subcore_barrier()
    pltpu.sync_copy(deg_sp.at[pl.ds(s * DEGS, DEGS)],
                    out_hbm.at[c, 0, pl.ds(s * DEGS, DEGS)])


_deg_call = pl.kernel(
    _deg_body,
    out_type=jax.ShapeDtypeStruct((NC, 1, DEGP), jnp.float32),
    mesh=_mesh,
    scratch_types=[
        pltpu.VMEM_SHARED((DEGP,), jnp.float32),
        pltpu.VMEM((RPW, CHUNK), jnp.int32),
        pltpu.VMEM((DEGS,), jnp.float32),
        pltpu.VMEM((CHUNK,), jnp.float32),
    ],
)


def _acc_body(y_hbm, src_hbm, dst_hbm, out_hbm,
              acc_sp, src_blk, dst_blk, rows_v0, rows_v1, rows_v2, rows_v3,
              gsem0, gsem1, gsem2, gsem3, ssem0, ssem1, ssem2, ssem3):
    c = lax.axis_index("c")
    s = lax.axis_index("s")
    w = s * NC + c
    d = rows_v0.shape[1]

    @pl.when(jnp.logical_and(c == 0, s < NS - 1))
    def _():
        pltpu.sync_copy(y_hbm.at[pl.ds(s * RT, RT)],
                        acc_sp.at[pl.ds(s * RT, RT)])

    @pl.when(jnp.logical_and(c == 0, s == NS - 1))
    def _():
        pltpu.sync_copy(y_hbm.at[pl.ds((NS - 1) * RT, RTL)],
                        acc_sp.at[pl.ds((NS - 1) * RT, RTL)])

    @pl.when(c != 0)
    def _():
        def zf(r, _):
            for kk in range(d // LANES):
                rows_v0[r, pl.ds(kk * LANES, LANES)] = (
                    jnp.zeros((LANES,), jnp.float32))
            return 0

        lax.fori_loop(0, CHUNK, zf, 0)
        start = s * RT
        end = jnp.minimum(start + RT, N)
        for k in range(RT // CHUNK + 1):
            base = jnp.minimum(start + k * CHUNK, end - CHUNK)
            pltpu.sync_copy(rows_v0, acc_sp.at[pl.ds(base, CHUNK)])


    gat = y_hbm
    plsc.subcore_barrier()

    rows = (rows_v0, rows_v1, rows_v2, rows_v3)
    gsem = (gsem0, gsem1, gsem2, gsem3)
    ssem = (ssem0, ssem1, ssem2, ssem3)

    def drain(buf, sem_):
        pltpu.make_async_copy(y_hbm.at[pl.ds(0, CHUNK)], buf, sem_).wait()

    def quarter(j, b, sguard, gguard):
        o = (b + 2) % 4
        if sguard is None:
            drain(rows[o], ssem[o])
        else:
            @pl.when(jnp.logical_not(sguard))
            def _():
                drain(rows[o], ssem[o])
        if gguard is None:
            pltpu.async_copy(gat.at[src_blk.at[j + 2]], rows[o], gsem[o])
        else:
            @pl.when(jnp.logical_not(gguard))
            def _():
                pltpu.async_copy(gat.at[src_blk.at[j + 2]], rows[o], gsem[o])
        drain(rows[b], gsem[b])
        pltpu.async_copy(rows[b], acc_sp.at[dst_blk.at[j]], ssem[b], add=True)

    nq = HRP // 4
    for p in range(RPW // HRP):
        pltpu.sync_copy(src_hbm.at[pl.ds(w * RPW + p * HRP, HRP)], src_blk)
        pltpu.sync_copy(dst_hbm.at[pl.ds(w * RPW + p * HRP, HRP)], dst_blk)
        pltpu.async_copy(gat.at[src_blk.at[0]], rows[0], gsem[0])
        pltpu.async_copy(gat.at[src_blk.at[1]], rows[1], gsem[1])

        def body(j0, _):
            quarter(4 * j0, 0, j0 == 0, None)
            quarter(4 * j0 + 1, 1, j0 == 0, None)
            quarter(4 * j0 + 2, 2, None, j0 == nq - 1)
            quarter(4 * j0 + 3, 3, None, j0 == nq - 1)
            return 0

        lax.fori_loop(0, nq, body, 0)
        drain(rows[2], ssem[2])
        drain(rows[3], ssem[3])
    plsc.subcore_barrier()

    @pl.when(s < NS - 1)
    def _():
        pltpu.sync_copy(acc_sp.at[pl.ds(s * RT, RT)],
                        out_hbm.at[c, pl.ds(s * RT, RT)])

    @pl.when(s == NS - 1)
    def _():
        pltpu.sync_copy(acc_sp.at[pl.ds((NS - 1) * RT, RTL)],
                        out_hbm.at[c, pl.ds((NS - 1) * RT, RTL)])


def _make_acc_call(d):
    return pl.kernel(
        _acc_body,
        out_type=jax.ShapeDtypeStruct((NC, N, d), jnp.float32),
        mesh=_mesh,
        scratch_types=[
            pltpu.VMEM_SHARED((ACCP, d), jnp.float32),
            pltpu.VMEM((HRP, CHUNK), jnp.int32),
            pltpu.VMEM((HRP, CHUNK), jnp.int32),
            pltpu.VMEM((CHUNK, d), jnp.float32),
            pltpu.VMEM((CHUNK, d), jnp.float32),
            pltpu.VMEM((CHUNK, d), jnp.float32),
            pltpu.VMEM((CHUNK, d), jnp.float32),
        ] + [pltpu.SemaphoreType.DMA] * 8,
    )


_acc_call_128 = _make_acc_call(F1)
_acc_call_p2 = _make_acc_call(F2P)

_MB = 1000
_GRID = N // _MB


def _dinv(degt):
    return lax.rsqrt(degt[:, 0:1] + degt[:, 1:2] + 1.0)


def _tc1a_body(x_ref, w_ref, xw_ref):
    xw_ref[...] = jnp.dot(x_ref[...], w_ref[...],
                          preferred_element_type=jnp.float32)


_tc1a = pl.pallas_call(
    _tc1a_body,
    grid=(_GRID,),
    in_specs=[
        pl.BlockSpec((_MB, F1), lambda i: (i, 0)),
        pl.BlockSpec((F1, F1), lambda i: (0, 0)),
    ],
    out_specs=pl.BlockSpec((_MB, F1), lambda i: (i, 0)),
    out_shape=jax.ShapeDtypeStruct((N, F1), jnp.float32),
)


def _tc1b_body(xw_ref, degt_ref, y1_ref):
    y1_ref[...] = xw_ref[...] * _dinv(degt_ref[...])


_tc1b = pl.pallas_call(
    _tc1b_body,
    grid=(_GRID,),
    in_specs=[
        pl.BlockSpec((_MB, F1), lambda i: (i, 0)),
        pl.BlockSpec((_MB, NC), lambda i: (i, 0)),
    ],
    out_specs=pl.BlockSpec((_MB, F1), lambda i: (i, 0)),
    out_shape=jax.ShapeDtypeStruct((N, F1), jnp.float32),
)


def _tc2_body(p0_ref, p1_ref, degt_ref, b1_ref, w2_ref, h_ref, y2_ref):
    dinv = _dinv(degt_ref[...])
    h = jnp.maximum((p0_ref[...] + p1_ref[...]) * dinv + b1_ref[...], 0.0)
    h_ref[...] = h
    y2_ref[...] = jnp.dot(h, w2_ref[...],
                          preferred_element_type=jnp.float32) * dinv


_tc2 = pl.pallas_call(
    _tc2_body,
    grid=(_GRID,),
    in_specs=[
        pl.BlockSpec((_MB, F1), lambda i: (i, 0)),
        pl.BlockSpec((_MB, F1), lambda i: (i, 0)),
        pl.BlockSpec((_MB, NC), lambda i: (i, 0)),
        pl.BlockSpec((1, F1), lambda i: (0, 0)),
        pl.BlockSpec((F1, F2P), lambda i: (0, 0)),
    ],
    out_specs=[
        pl.BlockSpec((_MB, F1), lambda i: (i, 0)),
        pl.BlockSpec((_MB, F2P), lambda i: (i, 0)),
    ],
    out_shape=[
        jax.ShapeDtypeStruct((N, F1), jnp.float32),
        jax.ShapeDtypeStruct((N, F2P), jnp.float32),
    ],
)


def _tc3_body(p0_ref, p1_ref, degt_ref, b2_ref, o_ref):
    o = (p0_ref[...] + p1_ref[...]) * _dinv(degt_ref[...])
    o_ref[...] = o[:, :F2] + b2_ref[...]


_tc3 = pl.pallas_call(
    _tc3_body,
    grid=(_GRID,),
    in_specs=[
        pl.BlockSpec((_MB, F2P), lambda i: (i, 0)),
        pl.BlockSpec((_MB, F2P), lambda i: (i, 0)),
        pl.BlockSpec((_MB, NC), lambda i: (i, 0)),
        pl.BlockSpec((1, F2), lambda i: (0, 0)),
    ],
    out_specs=pl.BlockSpec((_MB, F2), lambda i: (i, 0)),
    out_shape=jax.ShapeDtypeStruct((N, F2), jnp.float32),
)


def kernel(x, edge_index, W1, b1, W2, b2):
    ei = edge_index.astype(jnp.int32)
    pr = jnp.arange(EPAD, dtype=jnp.int32)
    srcp = jnp.concatenate([ei[0], pr % CHUNK]).reshape(EROWS, CHUNK)
    dstp = jnp.concatenate([ei[1], pr % CHUNK + N]).reshape(EROWS, CHUNK)

    xw = _tc1a(x, W1)
    deg_p = _deg_call(dstp).reshape(NC, DEGP)
    degt = deg_p[:, :N].T

    y1 = _tc1b(xw, degt)
    acc1 = _acc_call_128(y1, srcp, dstp)
    h, y2 = _tc2(acc1[0], acc1[1], degt, b1.reshape(1, F1),
                 jnp.pad(W2, ((0, 0), (0, F2P - F2))))
    acc2 = _acc_call_p2(y2, srcp, dstp)
    out = _tc3(acc2[0], acc2[1], degt, b2.reshape(1, F2))
    return h, out


# speedup vs baseline: 1.0346x; 1.0346x over previous
import functools

import jax
import jax.numpy as jnp
from jax import lax
from jax.experimental import pallas as pl
from jax.experimental.pallas import tpu as pltpu
from jax.experimental.pallas import tpu_sc as plsc

N = 10000
E = 320000
F1 = 128
F2 = 40
F2P = 128
LANES = 16
NC, NS = 2, 16
NW = NC * NS
CHUNK = 128
EROWS = 2560
EPAD = EROWS * CHUNK - E
RPW = EROWS // NW
HRP = RPW // 2
RT = 632
RTL = N - (NS - 1) * RT
DEGS = 640
DEGP = NS * DEGS
ACCP = N + CHUNK

_mesh = plsc.VectorSubcoreMesh(core_axis_name="c", subcore_axis_name="s")


def _deg_body(dst_hbm, out_hbm, deg_sp, dst_blk, zb, ones_v):
    c = lax.axis_index("c")
    s = lax.axis_index("s")
    w = s * NC + c

    def fill(i, _):
        zb[pl.ds(i * LANES, LANES)] = jnp.zeros((LANES,), jnp.float32)
        return 0

    lax.fori_loop(0, DEGS // LANES, fill, 0)

    def fill1(i, _):
        ones_v[pl.ds(i * LANES, LANES)] = jnp.ones((LANES,), jnp.float32)
        return 0

    lax.fori_loop(0, CHUNK // LANES, fill1, 0)

    pltpu.sync_copy(zb, deg_sp.at[pl.ds(s * DEGS, DEGS)])
    pltpu.sync_copy(dst_hbm.at[pl.ds(w * RPW, RPW)], dst_blk)
    plsc.subcore_barrier()

    def body(j, _):
        pltpu.sync_copy(ones_v, deg_sp.at[dst_blk.at[j]], add=True)
        return 0

    lax.fori_loop(0, RPW, body, 0)
    plsc.subcore_barrier()
    pltpu.sync_copy(deg_sp.at[pl.ds(s * DEGS, DEGS)],
                    out_hbm.at[c, 0, pl.ds(s * DEGS, DEGS)])


_deg_call = pl.kernel(
    _deg_body,
    out_type=jax.ShapeDtypeStruct((NC, 1, DEGP), jnp.float32),
    mesh=_mesh,
    scratch_types=[
        pltpu.VMEM_SHARED((DEGP,), jnp.float32),
        pltpu.VMEM((RPW, CHUNK), jnp.int32),
        pltpu.VMEM((DEGS,), jnp.float32),
        pltpu.VMEM((CHUNK,), jnp.float32),
    ],
)


def _acc_body(y_hbm, src_hbm, dst_hbm, out_hbm,
              acc_sp, src_blk, dst_blk, rows_v0, rows_v1,
              gsem0, gsem1, ssem0, ssem1):
    c = lax.axis_index("c")
    s = lax.axis_index("s")
    w = s * NC + c
    d = rows_v0.shape[1]

    @pl.when(jnp.logical_and(c == 0, s < NS - 1))
    def _():
        pltpu.sync_copy(y_hbm.at[pl.ds(s * RT, RT)],
                        acc_sp.at[pl.ds(s * RT, RT)])

    @pl.when(jnp.logical_and(c == 0, s == NS - 1))
    def _():
        pltpu.sync_copy(y_hbm.at[pl.ds((NS - 1) * RT, RTL)],
                        acc_sp.at[pl.ds((NS - 1) * RT, RTL)])

    @pl.when(c != 0)
    def _():
        def zf(r, _):
            for kk in range(d // LANES):
                rows_v0[r, pl.ds(kk * LANES, LANES)] = (
                    jnp.zeros((LANES,), jnp.float32))
            return 0

        lax.fori_loop(0, CHUNK, zf, 0)
        start = s * RT
        end = jnp.minimum(start + RT, N)
        for k in range(RT // CHUNK + 1):
            base = jnp.minimum(start + k * CHUNK, end - CHUNK)
            pltpu.sync_copy(rows_v0, acc_sp.at[pl.ds(base, CHUNK)])


    gat = y_hbm
    plsc.subcore_barrier()

    rows = (rows_v0, rows_v1)
    gsem = (gsem0, gsem1)
    ssem = (ssem0, ssem1)

    def drain(buf, sem_):
        pltpu.make_async_copy(y_hbm.at[pl.ds(0, CHUNK)], buf, sem_).wait()

    def half(j, b, first, last):
        o = 1 - b
        if first is None:
            drain(rows[o], ssem[o])
        else:
            @pl.when(jnp.logical_not(first))
            def _():
                drain(rows[o], ssem[o])
        if last is None:
            pltpu.async_copy(gat.at[src_blk.at[j + 1]], rows[o], gsem[o])
        else:
            @pl.when(jnp.logical_not(last))
            def _():
                pltpu.async_copy(gat.at[src_blk.at[j + 1]], rows[o], gsem[o])
        drain(rows[b], gsem[b])
        pltpu.async_copy(rows[b], acc_sp.at[dst_blk.at[j]], ssem[b], add=True)

    for p in range(RPW // HRP):
        pltpu.sync_copy(src_hbm.at[pl.ds(w * RPW + p * HRP, HRP)], src_blk)
        pltpu.sync_copy(dst_hbm.at[pl.ds(w * RPW + p * HRP, HRP)], dst_blk)
        pltpu.async_copy(gat.at[src_blk.at[0]], rows[0], gsem[0])

        def body(j0, _):
            half(2 * j0, 0, j0 == 0, None)
            half(2 * j0 + 1, 1, None, j0 == HRP // 2 - 1)
            return 0

        lax.fori_loop(0, HRP // 2, body, 0)
        drain(rows[1], ssem[1])
    plsc.subcore_barrier()

    @pl.when(s < NS - 1)
    def _():
        pltpu.sync_copy(acc_sp.at[pl.ds(s * RT, RT)],
                        out_hbm.at[c, pl.ds(s * RT, RT)])

    @pl.when(s == NS - 1)
    def _():
        pltpu.sync_copy(acc_sp.at[pl.ds((NS - 1) * RT, RTL)],
                        out_hbm.at[c, pl.ds((NS - 1) * RT, RTL)])


def _make_acc_call(d):
    return pl.kernel(
        _acc_body,
        out_type=jax.ShapeDtypeStruct((NC, N, d), jnp.float32),
        mesh=_mesh,
        scratch_types=[
            pltpu.VMEM_SHARED((ACCP, d), jnp.float32),
            pltpu.VMEM((HRP, CHUNK), jnp.int32),
            pltpu.VMEM((HRP, CHUNK), jnp.int32),
            pltpu.VMEM((CHUNK, d), jnp.float32),
            pltpu.VMEM((CHUNK, d), jnp.float32),
            pltpu.SemaphoreType.DMA,
            pltpu.SemaphoreType.DMA,
            pltpu.SemaphoreType.DMA,
            pltpu.SemaphoreType.DMA,
        ],
    )


_acc_call_128 = _make_acc_call(F1)
_acc_call_p2 = _make_acc_call(F2P)

_MB = 1000
_GRID = N // _MB


def _dinv(degt):
    return lax.rsqrt(degt[:, 0:1] + degt[:, 1:2] + 1.0)


def _tc1a_body(x_ref, w_ref, xw_ref):
    xw_ref[...] = jnp.dot(x_ref[...], w_ref[...],
                          preferred_element_type=jnp.float32)


_tc1a = pl.pallas_call(
    _tc1a_body,
    grid=(_GRID,),
    in_specs=[
        pl.BlockSpec((_MB, F1), lambda i: (i, 0)),
        pl.BlockSpec((F1, F1), lambda i: (0, 0)),
    ],
    out_specs=pl.BlockSpec((_MB, F1), lambda i: (i, 0)),
    out_shape=jax.ShapeDtypeStruct((N, F1), jnp.float32),
)


def _tc1b_body(xw_ref, degt_ref, y1_ref):
    y1_ref[...] = xw_ref[...] * _dinv(degt_ref[...])


_tc1b = pl.pallas_call(
    _tc1b_body,
    grid=(_GRID,),
    in_specs=[
        pl.BlockSpec((_MB, F1), lambda i: (i, 0)),
        pl.BlockSpec((_MB, NC), lambda i: (i, 0)),
    ],
    out_specs=pl.BlockSpec((_MB, F1), lambda i: (i, 0)),
    out_shape=jax.ShapeDtypeStruct((N, F1), jnp.float32),
)


def _tc2_body(p0_ref, p1_ref, degt_ref, b1_ref, w2_ref, h_ref, y2_ref):
    dinv = _dinv(degt_ref[...])
    h = jnp.maximum((p0_ref[...] + p1_ref[...]) * dinv + b1_ref[...], 0.0)
    h_ref[...] = h
    y2_ref[...] = jnp.dot(h, w2_ref[...],
                          preferred_element_type=jnp.float32) * dinv


_tc2 = pl.pallas_call(
    _tc2_body,
    grid=(_GRID,),
    in_specs=[
        pl.BlockSpec((_MB, F1), lambda i: (i, 0)),
        pl.BlockSpec((_MB, F1), lambda i: (i, 0)),
        pl.BlockSpec((_MB, NC), lambda i: (i, 0)),
        pl.BlockSpec((1, F1), lambda i: (0, 0)),
        pl.BlockSpec((F1, F2P), lambda i: (0, 0)),
    ],
    out_specs=[
        pl.BlockSpec((_MB, F1), lambda i: (i, 0)),
        pl.BlockSpec((_MB, F2P), lambda i: (i, 0)),
    ],
    out_shape=[
        jax.ShapeDtypeStruct((N, F1), jnp.float32),
        jax.ShapeDtypeStruct((N, F2P), jnp.float32),
    ],
)


def _tc3_body(p0_ref, p1_ref, degt_ref, b2_ref, o_ref):
    o = (p0_ref[...] + p1_ref[...]) * _dinv(degt_ref[...])
    o_ref[...] = o[:, :F2] + b2_ref[...]


_tc3 = pl.pallas_call(
    _tc3_body,
    grid=(_GRID,),
    in_specs=[
        pl.BlockSpec((_MB, F2P), lambda i: (i, 0)),
        pl.BlockSpec((_MB, F2P), lambda i: (i, 0)),
        pl.BlockSpec((_MB, NC), lambda i: (i, 0)),
        pl.BlockSpec((1, F2), lambda i: (0, 0)),
    ],
    out_specs=pl.BlockSpec((_MB, F2), lambda i: (i, 0)),
    out_shape=jax.ShapeDtypeStruct((N, F2), jnp.float32),
)


def kernel(x, edge_index, W1, b1, W2, b2):
    ei = edge_index.astype(jnp.int32)
    pr = jnp.arange(EPAD, dtype=jnp.int32)
    srcp = jnp.concatenate([ei[0], pr % CHUNK]).reshape(EROWS, CHUNK)
    dstp = jnp.concatenate([ei[1], pr % CHUNK + N]).reshape(EROWS, CHUNK)

    xw = _tc1a(x, W1)
    deg_p = _deg_call(dstp).reshape(NC, DEGP)
    degt = deg_p[:, :N].T

    y1 = _tc1b(xw, degt)
    acc1 = _acc_call_128(y1, srcp, dstp)
    h, y2 = _tc2(acc1[0], acc1[1], degt, b1.reshape(1, F1),
                 jnp.pad(W2, ((0, 0), (0, F2P - F2))))
    acc2 = _acc_call_p2(y2, srcp, dstp)
    out = _tc3(acc2[0], acc2[1], degt, b2.reshape(1, F2))
    return h, out
